# Initial kernel scaffold; baseline (speedup 1.0000x reference)
import jax
import jax.numpy as jnp
from jax.experimental import pallas as pl


def kernel(x_member, x_provider, W1, b1, W2, b2, Wd, bd, edge_index):
    raise NotImplementedError("write your pallas kernel here")


# speedup vs baseline: 24.1581x; 24.1581x over previous
import functools

import jax
import jax.numpy as jnp
from jax import lax
from jax.experimental import pallas as pl
from jax.experimental.pallas import tpu as pltpu
from jax.experimental.pallas import tpu_sc as plsc

NM_, NP_, E_ = 8000, 2000, 320000
COLS = 2000
QROWS = 500
Q_ELEMS = QROWS * COLS
WRITE_PER_SUB = 62592
Q_OUT = 16 * WRITE_PER_SUB
SP_SIZE = Q_OUT
GARB = Q_ELEMS
CHUNK = 1280
ROWS_PER_CHUNK = CHUNK // 128
VECS_PER_CHUNK = CHUNK // 16
ED_MAIN = 20480
ZB = 16384


def _sc_build_k(pidx, midx):
    mesh = plsc.VectorSubcoreMesh(core_axis_name="c", subcore_axis_name="s")

    @functools.partial(
        pl.kernel,
        out_type=jax.ShapeDtypeStruct((4 * Q_OUT,), jnp.float32),
        mesh=mesh,
        scratch_types=[
            pltpu.VMEM((CHUNK,), jnp.int32),
            pltpu.VMEM((CHUNK,), jnp.int32),
            pltpu.VMEM((ROWS_PER_CHUNK, 128), jnp.int32),
            pltpu.VMEM((128,), jnp.float32),
            pltpu.VMEM((ZB,), jnp.float32),
            pltpu.VMEM_SHARED((SP_SIZE,), jnp.float32),
            pltpu.SemaphoreType.DMA,
        ],
    )
    def build(zeros_hbm, ones_hbm, pidx_hbm, midx_hbm, kout_hbm,
              mbuf, pbuf, idxbuf, ones, zbuf, ksp, sem):
        c = lax.axis_index("c")
        s = lax.axis_index("s")

        pltpu.sync_copy(ones_hbm, ones)

        zbase = s * WRITE_PER_SUB
        nchunks = jnp.where(s == 15, (E_ - 15 * ED_MAIN) // CHUNK,
                            ED_MAIN // CHUNK)
        ebase = s * ED_MAIN
        nz_full = WRITE_PER_SUB // ZB
        rem = WRITE_PER_SUB % ZB

        for q in range(2):
            lo = c * 1000 + q * QROWS

            pltpu.sync_copy(zeros_hbm, zbuf)
            for k in range(nz_full):
                pltpu.sync_copy(zbuf, ksp.at[pl.ds(zbase + k * ZB, ZB)])
            if rem:
                pltpu.sync_copy(zbuf.at[pl.ds(0, rem)],
                                ksp.at[pl.ds(zbase + nz_full * ZB, rem)])
            plsc.subcore_barrier()

            def chunk_body(ci, carry):
                base = ebase + ci * CHUNK
                pltpu.sync_copy(midx_hbm.at[pl.ds(base, CHUNK)], mbuf)
                pltpu.sync_copy(pidx_hbm.at[pl.ds(base, CHUNK)], pbuf)
                for v in range(VECS_PER_CHUNK):
                    m = mbuf[pl.ds(v * 16, 16)]
                    p = pbuf[pl.ds(v * 16, 16)]
                    rel = m - lo
                    ok = (rel >= 0) & (rel < QROWS)
                    flat = jnp.where(ok, rel * COLS + p, GARB)
                    idxbuf[v // 8, pl.ds((v % 8) * 16, 16)] = flat
                copies = [
                    pltpu.async_copy(ones, ksp.at[idxbuf.at[r]], sem, add=True)
                    for r in range(ROWS_PER_CHUNK)
                ]
                for cp in copies:
                    cp.wait()
                return carry

            lax.fori_loop(0, nchunks, chunk_body, 0)
            plsc.subcore_barrier()

            obase = (c * 2 + q) * Q_OUT + zbase
            for k in range(nz_full):
                pltpu.sync_copy(ksp.at[pl.ds(zbase + k * ZB, ZB)], zbuf)
                pltpu.sync_copy(zbuf, kout_hbm.at[pl.ds(obase + k * ZB, ZB)])
            if rem:
                off = nz_full * ZB
                pltpu.sync_copy(ksp.at[pl.ds(zbase + off, rem)],
                                zbuf.at[pl.ds(0, rem)])
                pltpu.sync_copy(zbuf.at[pl.ds(0, rem)],
                                kout_hbm.at[pl.ds(obase + off, rem)])
            if q == 0:
                plsc.subcore_barrier()

    return build(jnp.zeros((ZB,), jnp.float32), jnp.ones((128,), jnp.float32),
                 pidx, midx)


def _dot(a, b):
    return lax.dot_general(a, b, (((1,), (0,)), ((), ())),
                           preferred_element_type=jnp.float32,
                           precision=lax.Precision.HIGHEST)


def _dot_t(a, b):
    return lax.dot_general(a, b, (((0,), (0,)), ((), ())),
                           preferred_element_type=jnp.float32,
                           precision=lax.Precision.HIGHEST)


KB = 200
NKB = COLS // KB


def _k_sums(K):
    def body(K_ref, rs_ref, cs_ref):
        i = pl.program_id(0)
        Kb = K_ref[...]
        rs_ref[...] = jnp.sum(Kb, axis=1)[None, None, :]

        @pl.when(i == 0)
        def _():
            cs_ref[...] = jnp.zeros((COLS,), jnp.float32)

        cs_ref[...] += jnp.sum(Kb, axis=0)

    return pl.pallas_call(
        body,
        grid=(NKB,),
        in_specs=[pl.BlockSpec((KB, COLS), lambda i: (i, 0))],
        out_specs=(pl.BlockSpec((1, 1, KB), lambda i: (i, 0, 0)),
                   pl.BlockSpec((COLS,), lambda i: (0,))),
        out_shape=(jax.ShapeDtypeStruct((NKB, 1, KB), jnp.float32),
                   jax.ShapeDtypeStruct((COLS,), jnp.float32)),
    )(K)


def _k_scale(K, rs, cs):
    def body(K_ref, rs_ref, cs_ref, Ks_ref):
        dB = lax.rsqrt(rs_ref[...][0, 0] + 1.0)
        dA = lax.rsqrt(cs_ref[...] + 1.0)
        Ks_ref[...] = dB[:, None] * K_ref[...] * dA[None, :]

    return pl.pallas_call(
        body,
        grid=(NKB,),
        in_specs=[pl.BlockSpec((KB, COLS), lambda i: (i, 0)),
                  pl.BlockSpec((1, 1, KB), lambda i: (i, 0, 0)),
                  pl.BlockSpec((COLS,), lambda i: (0,))],
        out_specs=pl.BlockSpec((KB, COLS), lambda i: (i, 0)),
        out_shape=jax.ShapeDtypeStruct((COLS, COLS), jnp.float32),
    )(K, rs, cs)


def _matmul(x, W):
    def body(x_ref, W_ref, o_ref):
        o_ref[...] = _dot(x_ref[...], W_ref[...])

    n, d = x.shape[0], W.shape[1]
    return pl.pallas_call(
        body,
        out_shape=jax.ShapeDtypeStruct((n, d), jnp.float32),
    )(x, W)


def _gcn_layer(Ks, hA, hB, rs, cs, b):
    d = hA.shape[1]

    def body(Ks_ref, hA_ref, hB_ref, rs_ref, cs_ref, b_ref, oA_ref, oB_ref):
        i = pl.program_id(0)
        Ks = Ks_ref[...]
        hA = hA_ref[...]
        dB2 = 1.0 / (rs_ref[...][0, 0] + 1.0)
        b = b_ref[...][None, :]
        oB_ref[...] = _dot(Ks, hA) + dB2[:, None] * hB_ref[...] + b

        @pl.when(i == 0)
        def _():
            dA2 = 1.0 / (cs_ref[...] + 1.0)
            oA_ref[...] = dA2[:, None] * hA + b

        oA_ref[...] += _dot_t(Ks, hB_ref[...])

    return pl.pallas_call(
        body,
        grid=(NKB,),
        in_specs=[pl.BlockSpec((KB, COLS), lambda i: (i, 0)),
                  pl.BlockSpec((COLS, d), lambda i: (0, 0)),
                  pl.BlockSpec((KB, d), lambda i: (i, 0)),
                  pl.BlockSpec((1, 1, KB), lambda i: (i, 0, 0)),
                  pl.BlockSpec((COLS,), lambda i: (0,)),
                  pl.BlockSpec((d,), lambda i: (0,))],
        out_specs=(pl.BlockSpec((COLS, d), lambda i: (0, 0)),
                   pl.BlockSpec((KB, d), lambda i: (i, 0))),
        out_shape=(jax.ShapeDtypeStruct((COLS, d), jnp.float32),
                   jax.ShapeDtypeStruct((COLS, d), jnp.float32)),
    )(Ks, hA, hB, rs, cs, b)


def _relu_matmul(x, W):
    def body(x_ref, W_ref, o_ref):
        o_ref[...] = _dot(jnp.maximum(x_ref[...], 0.0), W_ref[...])

    n, d = x.shape[0], W.shape[1]
    return pl.pallas_call(
        body,
        out_shape=jax.ShapeDtypeStruct((n, d), jnp.float32),
    )(x, W)


def _mid_mlp(h, b1, W2, b2, Wd, bd):
    def body(h_ref, b1_ref, W2_ref, b2_ref, Wd_ref, bd_ref, o_ref):
        r = jnp.maximum(h_ref[...] + b1_ref[...][None, :], 0.0)
        z = _dot(r, W2_ref[...]) + b2_ref[...][None, :]
        o_ref[...] = _dot(z, Wd_ref[...]) + bd_ref[...][None, :]

    n = h.shape[0]
    return pl.pallas_call(
        body,
        out_shape=jax.ShapeDtypeStruct((n, 128), jnp.float32),
    )(h, b1, W2, b2, Wd, bd)


def _matmul_bias(x, W, b):
    def body(x_ref, W_ref, b_ref, o_ref):
        o_ref[...] = _dot(x_ref[...], W_ref[...]) + b_ref[...][None, :]

    n, d = x.shape[0], W.shape[1]
    return pl.pallas_call(
        body,
        out_shape=jax.ShapeDtypeStruct((n, d), jnp.float32),
    )(x, W, b)


def kernel(x_member, x_provider, W1, b1, W2, b2, Wd, bd, edge_index):
    pidx = edge_index[0]
    midx = edge_index[1]
    k_flat = _sc_build_k(pidx, midx)
    K = jnp.concatenate(
        [k_flat[i * Q_OUT:i * Q_OUT + Q_ELEMS] for i in range(4)]
    ).reshape(COLS, COLS)

    rs, cs = _k_sums(K)
    Ks = _k_scale(K, rs, cs)

    xp_pad = jnp.pad(x_provider, ((0, 0), (0, 128 - x_provider.shape[1])))
    x10 = jnp.concatenate([x_member, xp_pad], axis=0)
    h10 = _matmul(x10, W1)
    hA, hM, hB = h10[:NP_], h10[NP_:NM_], h10[NM_:]

    a1A, a1B = _gcn_layer(Ks, hA, hB, rs, cs, b1)
    gAB = _relu_matmul(jnp.concatenate([a1A, a1B], axis=0), W2)
    zA, zB = _gcn_layer(Ks, gAB[:NP_], gAB[NP_:], rs, cs, b2)

    out_AB = _matmul_bias(jnp.concatenate([zA, zB], axis=0), Wd, bd)
    out_mid = _mid_mlp(hM, b1, W2, b2, Wd, bd)

    xhat_m = jnp.concatenate([out_AB[:NP_], out_mid], axis=0)
    xhat_p = out_AB[NP_:]
    edge_logits = jnp.zeros((edge_index.shape[1],), dtype=jnp.float32)
    return (xhat_m, xhat_p[:, : x_provider.shape[1]], edge_logits)


# speedup vs baseline: 53.5785x; 2.2178x over previous
import functools

import jax
import jax.numpy as jnp
from jax import lax
from jax.experimental import pallas as pl
from jax.experimental.pallas import tpu as pltpu
from jax.experimental.pallas import tpu_sc as plsc

NM_, NP_, E_ = 8000, 2000, 320000
COLS = 2000
PCOLS = COLS // 2
CORE_ELEMS = 1000 * PCOLS
WRITE_PER_SUB = 62592
SP_SIZE = 16 * WRITE_PER_SUB
GARB = CORE_ELEMS
CHUNK = 1280
ROWS_PER_CHUNK = CHUNK // 128
VECS_PER_CHUNK = CHUNK // 16
ED_MAIN = 16 * CHUNK


def _sc_build_k(pidx, midx):
    mesh = plsc.VectorSubcoreMesh(core_axis_name="c", subcore_axis_name="s")

    @functools.partial(
        pl.kernel,
        out_type=jax.ShapeDtypeStruct((2 * SP_SIZE,), jnp.int32),
        mesh=mesh,
        scratch_types=[
            pltpu.VMEM((CHUNK,), jnp.int32),
            pltpu.VMEM((CHUNK,), jnp.int32),
            pltpu.VMEM((ROWS_PER_CHUNK, 128), jnp.int32),
            pltpu.VMEM((ROWS_PER_CHUNK, 128), jnp.int32),
            pltpu.VMEM((2, 8192), jnp.int32),
            pltpu.VMEM_SHARED((SP_SIZE,), jnp.int32),
            pltpu.SemaphoreType.DMA,
            pltpu.SemaphoreType.DMA,
        ],
    )
    def build(zeros_hbm, pidx_hbm, midx_hbm, kout_hbm,
              mbuf, pbuf, idxbuf, valbuf, bnc, ksp, sem_ld, sem_sc):
        c = lax.axis_index("c")
        s = lax.axis_index("s")
        lo = c * 1000
        iota16 = lax.iota(jnp.int32, 16)

        zbase = s * WRITE_PER_SUB
        pltpu.sync_copy(zeros_hbm, bnc.at[0])
        zcopies = []
        for k in range(7):
            zcopies.append(pltpu.async_copy(
                bnc.at[0], ksp.at[pl.ds(zbase + k * 8192, 8192)], sem_ld))
        zcopies.append(pltpu.async_copy(
            bnc.at[0, pl.ds(0, 5248)],
            ksp.at[pl.ds(zbase + 7 * 8192, 5248)], sem_ld))
        for cp in zcopies:
            cp.wait()
        plsc.subcore_barrier()

        nchunks = jnp.where(s < 10, 16, 15)
        ebase = s * ED_MAIN - jnp.maximum(s - 10, 0) * CHUNK

        def chunk_body(ci, carry):
            base = ebase + ci * CHUNK
            pltpu.sync_copy(midx_hbm.at[pl.ds(base, CHUNK)], mbuf)
            pltpu.sync_copy(pidx_hbm.at[pl.ds(base, CHUNK)], pbuf)
            for v in range(VECS_PER_CHUNK):
                m = mbuf[pl.ds(v * 16, 16)]
                p = pbuf[pl.ds(v * 16, 16)]
                rel = m - lo
                ok = (rel >= 0) & (rel < 1000)
                word = rel * PCOLS + lax.shift_right_logical(p, 1)
                flat = jnp.where(ok, word, GARB + iota16)
                val = jnp.where(jnp.bitwise_and(p, 1) == 1, 65536, 1)
                idxbuf[v // 8, pl.ds((v % 8) * 16, 16)] = flat
                valbuf[v // 8, pl.ds((v % 8) * 16, 16)] = val
            copies = [
                pltpu.async_copy(valbuf.at[r], ksp.at[idxbuf.at[r]], sem_sc,
                                 add=True)
                for r in range(ROWS_PER_CHUNK)
            ]
            for cp in copies:
                cp.wait()
            return carry

        lax.fori_loop(0, nchunks, chunk_body, 0)
        plsc.subcore_barrier()

        obase = c * SP_SIZE + zbase
        outs = [None, None]
        for k in range(8):
            sz = 8192 if k < 7 else 5248
            b = k % 2
            if outs[b] is not None:
                outs[b].wait()
            pltpu.sync_copy(ksp.at[pl.ds(zbase + k * 8192, sz)],
                            bnc.at[b, pl.ds(0, sz)])
            outs[b] = pltpu.async_copy(
                bnc.at[b, pl.ds(0, sz)],
                kout_hbm.at[pl.ds(obase + k * 8192, sz)], sem_sc)
        for cp in outs:
            cp.wait()

    return build(jnp.zeros((8192,), jnp.int32), pidx, midx)


def _unpack_k(k_flat):
    packed = jnp.concatenate(
        [k_flat[:CORE_ELEMS], k_flat[SP_SIZE:SP_SIZE + CORE_ELEMS]]
    ).reshape(COLS, PCOLS)
    low = jnp.bitwise_and(packed, 65535)
    high = lax.shift_right_logical(packed, 16)
    return jnp.stack([low, high], axis=2).reshape(COLS, COLS).astype(jnp.float32)


def _dot(a, b):
    return lax.dot_general(a, b, (((1,), (0,)), ((), ())),
                           preferred_element_type=jnp.float32,
                           precision=lax.Precision.HIGHEST)


def _dot_t(a, b):
    return lax.dot_general(a, b, (((0,), (0,)), ((), ())),
                           preferred_element_type=jnp.float32,
                           precision=lax.Precision.HIGHEST)


KB = 200
NKB = COLS // KB


def _k_sums(K):
    def body(K_ref, rs_ref, cs_ref):
        i = pl.program_id(0)
        Kb = K_ref[...]
        rs_ref[...] = jnp.sum(Kb, axis=1)[None, None, :]

        @pl.when(i == 0)
        def _():
            cs_ref[...] = jnp.zeros((COLS,), jnp.float32)

        cs_ref[...] += jnp.sum(Kb, axis=0)

    return pl.pallas_call(
        body,
        grid=(NKB,),
        in_specs=[pl.BlockSpec((KB, COLS), lambda i: (i, 0))],
        out_specs=(pl.BlockSpec((1, 1, KB), lambda i: (i, 0, 0)),
                   pl.BlockSpec((COLS,), lambda i: (0,))),
        out_shape=(jax.ShapeDtypeStruct((NKB, 1, KB), jnp.float32),
                   jax.ShapeDtypeStruct((COLS,), jnp.float32)),
    )(K)


def _k_scale(K, rs, cs):
    def body(K_ref, rs_ref, cs_ref, Ks_ref):
        dB = lax.rsqrt(rs_ref[...][0, 0] + 1.0)
        dA = lax.rsqrt(cs_ref[...] + 1.0)
        Ks_ref[...] = dB[:, None] * K_ref[...] * dA[None, :]

    return pl.pallas_call(
        body,
        grid=(NKB,),
        in_specs=[pl.BlockSpec((KB, COLS), lambda i: (i, 0)),
                  pl.BlockSpec((1, 1, KB), lambda i: (i, 0, 0)),
                  pl.BlockSpec((COLS,), lambda i: (0,))],
        out_specs=pl.BlockSpec((KB, COLS), lambda i: (i, 0)),
        out_shape=jax.ShapeDtypeStruct((COLS, COLS), jnp.float32),
    )(K, rs, cs)


def _matmul(x, W):
    def body(x_ref, W_ref, o_ref):
        o_ref[...] = _dot(x_ref[...], W_ref[...])

    n, d = x.shape[0], W.shape[1]
    return pl.pallas_call(
        body,
        out_shape=jax.ShapeDtypeStruct((n, d), jnp.float32),
    )(x, W)


def _gcn_layer(Ks, hA, hB, rs, cs, b):
    d = hA.shape[1]

    def body(Ks_ref, hA_ref, hB_ref, rs_ref, cs_ref, b_ref, oA_ref, oB_ref):
        i = pl.program_id(0)
        Ks = Ks_ref[...]
        hA = hA_ref[...]
        dB2 = 1.0 / (rs_ref[...][0, 0] + 1.0)
        b = b_ref[...][None, :]
        oB_ref[...] = _dot(Ks, hA) + dB2[:, None] * hB_ref[...] + b

        @pl.when(i == 0)
        def _():
            dA2 = 1.0 / (cs_ref[...] + 1.0)
            oA_ref[...] = dA2[:, None] * hA + b

        oA_ref[...] += _dot_t(Ks, hB_ref[...])

    return pl.pallas_call(
        body,
        grid=(NKB,),
        in_specs=[pl.BlockSpec((KB, COLS), lambda i: (i, 0)),
                  pl.BlockSpec((COLS, d), lambda i: (0, 0)),
                  pl.BlockSpec((KB, d), lambda i: (i, 0)),
                  pl.BlockSpec((1, 1, KB), lambda i: (i, 0, 0)),
                  pl.BlockSpec((COLS,), lambda i: (0,)),
                  pl.BlockSpec((d,), lambda i: (0,))],
        out_specs=(pl.BlockSpec((COLS, d), lambda i: (0, 0)),
                   pl.BlockSpec((KB, d), lambda i: (i, 0))),
        out_shape=(jax.ShapeDtypeStruct((COLS, d), jnp.float32),
                   jax.ShapeDtypeStruct((COLS, d), jnp.float32)),
    )(Ks, hA, hB, rs, cs, b)


def _relu_matmul(x, W):
    def body(x_ref, W_ref, o_ref):
        o_ref[...] = _dot(jnp.maximum(x_ref[...], 0.0), W_ref[...])

    n, d = x.shape[0], W.shape[1]
    return pl.pallas_call(
        body,
        out_shape=jax.ShapeDtypeStruct((n, d), jnp.float32),
    )(x, W)


def _mid_mlp(h, b1, W2, b2, Wd, bd):
    def body(h_ref, b1_ref, W2_ref, b2_ref, Wd_ref, bd_ref, o_ref):
        r = jnp.maximum(h_ref[...] + b1_ref[...][None, :], 0.0)
        z = _dot(r, W2_ref[...]) + b2_ref[...][None, :]
        o_ref[...] = _dot(z, Wd_ref[...]) + bd_ref[...][None, :]

    n = h.shape[0]
    return pl.pallas_call(
        body,
        out_shape=jax.ShapeDtypeStruct((n, 128), jnp.float32),
    )(h, b1, W2, b2, Wd, bd)


def _matmul_bias(x, W, b):
    def body(x_ref, W_ref, b_ref, o_ref):
        o_ref[...] = _dot(x_ref[...], W_ref[...]) + b_ref[...][None, :]

    n, d = x.shape[0], W.shape[1]
    return pl.pallas_call(
        body,
        out_shape=jax.ShapeDtypeStruct((n, d), jnp.float32),
    )(x, W, b)


def kernel(x_member, x_provider, W1, b1, W2, b2, Wd, bd, edge_index):
    pidx = edge_index[0]
    midx = edge_index[1]
    k_flat = _sc_build_k(pidx, midx)
    K = _unpack_k(k_flat)

    rs, cs = _k_sums(K)
    Ks = _k_scale(K, rs, cs)

    xp_pad = jnp.pad(x_provider, ((0, 0), (0, 128 - x_provider.shape[1])))
    x10 = jnp.concatenate([x_member, xp_pad], axis=0)
    h10 = _matmul(x10, W1)
    hA, hM, hB = h10[:NP_], h10[NP_:NM_], h10[NM_:]

    a1A, a1B = _gcn_layer(Ks, hA, hB, rs, cs, b1)
    gAB = _relu_matmul(jnp.concatenate([a1A, a1B], axis=0), W2)
    zA, zB = _gcn_layer(Ks, gAB[:NP_], gAB[NP_:], rs, cs, b2)

    out_AB = _matmul_bias(jnp.concatenate([zA, zB], axis=0), Wd, bd)
    out_mid = _mid_mlp(hM, b1, W2, b2, Wd, bd)

    xhat_m = jnp.concatenate([out_AB[:NP_], out_mid], axis=0)
    xhat_p = out_AB[NP_:]
    edge_logits = jnp.zeros((edge_index.shape[1],), dtype=jnp.float32)
    return (xhat_m, xhat_p[:, : x_provider.shape[1]], edge_logits)


# speedup vs baseline: 95.1466x; 1.7758x over previous
import functools

import jax
import jax.numpy as jnp
from jax import lax
from jax.experimental import pallas as pl
from jax.experimental.pallas import tpu as pltpu
from jax.experimental.pallas import tpu_sc as plsc

NM_, NP_, E_ = 8000, 2000, 320000
PC = 1024
CORE_ELEMS = 1000 * PC
GARB = CORE_ELEMS
SP_SIZE = CORE_ELEMS + 128
WRITE_PER_SUB = CORE_ELEMS // 16
CHUNK = 1280
ROWS_PER_CHUNK = CHUNK // 128
VECS_PER_CHUNK = CHUNK // 16
ED_MAIN = 16 * CHUNK
PCD = 2 * PC


def _sc_build_k(pidx, midx):
    mesh = plsc.VectorSubcoreMesh(core_axis_name="c", subcore_axis_name="s")

    @functools.partial(
        pl.kernel,
        out_type=jax.ShapeDtypeStruct((2 * CORE_ELEMS,), jnp.int32),
        mesh=mesh,
        scratch_types=[
            pltpu.VMEM((CHUNK,), jnp.int32),
            pltpu.VMEM((CHUNK,), jnp.int32),
            pltpu.VMEM((ROWS_PER_CHUNK, 128), jnp.int32),
            pltpu.VMEM((ROWS_PER_CHUNK, 128), jnp.int32),
            pltpu.VMEM((2, 8192), jnp.int32),
            pltpu.VMEM_SHARED((SP_SIZE,), jnp.int32),
            pltpu.SemaphoreType.DMA,
            pltpu.SemaphoreType.DMA,
        ],
    )
    def build(zeros_hbm, pidx_hbm, midx_hbm, kout_hbm,
              mbuf, pbuf, idxbuf, valbuf, bnc, ksp, sem_ld, sem_sc):
        c = lax.axis_index("c")
        s = lax.axis_index("s")
        lo = c * 1000
        iota16 = lax.iota(jnp.int32, 16)

        zbase = s * WRITE_PER_SUB
        pltpu.sync_copy(zeros_hbm, bnc.at[0])
        zcopies = [
            pltpu.async_copy(bnc.at[0], ksp.at[pl.ds(zbase + k * 8192, 8192)],
                             sem_ld)
            for k in range(7)
        ]
        zcopies.append(pltpu.async_copy(
            bnc.at[0, pl.ds(0, 6656)],
            ksp.at[pl.ds(zbase + 7 * 8192, 6656)], sem_ld))
        for cp in zcopies:
            cp.wait()

        @pl.when(s == 0)
        def _():
            pltpu.sync_copy(bnc.at[0, pl.ds(0, 128)],
                            ksp.at[pl.ds(GARB, 128)])

        plsc.subcore_barrier()

        nchunks = jnp.where(s < 10, 16, 15)
        ebase = s * ED_MAIN - jnp.maximum(s - 10, 0) * CHUNK

        def chunk_body(ci, carry):
            base = ebase + ci * CHUNK
            ld_m = pltpu.async_copy(midx_hbm.at[pl.ds(base, CHUNK)], mbuf,
                                    sem_ld)
            ld_p = pltpu.async_copy(pidx_hbm.at[pl.ds(base, CHUNK)], pbuf,
                                    sem_ld)
            ld_m.wait()
            ld_p.wait()
            for v in range(VECS_PER_CHUNK):
                m = mbuf[pl.ds(v * 16, 16)]
                p = pbuf[pl.ds(v * 16, 16)]
                rel = m - lo
                ok = (rel >= 0) & (rel < 1000)
                word = rel * PC + lax.shift_right_logical(p, 1)
                flat = jnp.where(ok, word, GARB + iota16)
                val = jnp.where(jnp.bitwise_and(p, 1) == 1, 65536, 1)
                idxbuf[v // 8, pl.ds((v % 8) * 16, 16)] = flat
                valbuf[v // 8, pl.ds((v % 8) * 16, 16)] = val
            copies = [
                pltpu.async_copy(valbuf.at[r], ksp.at[idxbuf.at[r]], sem_sc,
                                 add=True)
                for r in range(ROWS_PER_CHUNK)
            ]
            for cp in copies:
                cp.wait()
            return carry

        lax.fori_loop(0, nchunks, chunk_body, 0)
        plsc.subcore_barrier()

        obase = c * CORE_ELEMS + zbase
        outs = [None, None]
        for k in range(8):
            sz = 8192 if k < 7 else 6656
            b = k % 2
            if outs[b] is not None:
                outs[b].wait()
            pltpu.sync_copy(ksp.at[pl.ds(zbase + k * 8192, sz)],
                            bnc.at[b, pl.ds(0, sz)])
            outs[b] = pltpu.async_copy(
                bnc.at[b, pl.ds(0, sz)],
                kout_hbm.at[pl.ds(obase + k * 8192, sz)], sem_sc)
        for cp in outs:
            cp.wait()

    return build(jnp.zeros((8192,), jnp.int32), pidx, midx)


def _dot(a, b):
    return lax.dot_general(a, b, (((1,), (0,)), ((), ())),
                           preferred_element_type=jnp.float32,
                           precision=lax.Precision.HIGHEST)


def _dot_t(a, b):
    return lax.dot_general(a, b, (((0,), (0,)), ((), ())),
                           preferred_element_type=jnp.float32,
                           precision=lax.Precision.HIGHEST)


KB = 200
NKB = 2000 // KB


def _unpack(pk):
    low = jnp.bitwise_and(pk, 65535).astype(jnp.float32)
    high = lax.shift_right_logical(pk, 16).astype(jnp.float32)
    return low, high


def _k_sums(Kp):
    def body(Kp_ref, rs_ref, cs_ref):
        i = pl.program_id(0)
        low, high = _unpack(Kp_ref[...])
        rs_ref[...] = jnp.sum(low + high, axis=1)[None, None, :]

        @pl.when(i == 0)
        def _():
            cs_ref[...] = jnp.zeros((PCD,), jnp.float32)

        cs_ref[0:PC] += jnp.sum(low, axis=0)
        cs_ref[PC:PCD] += jnp.sum(high, axis=0)

    return pl.pallas_call(
        body,
        grid=(NKB,),
        in_specs=[pl.BlockSpec((KB, PC), lambda i: (i, 0))],
        out_specs=(pl.BlockSpec((1, 1, KB), lambda i: (i, 0, 0)),
                   pl.BlockSpec((PCD,), lambda i: (0,))),
        out_shape=(jax.ShapeDtypeStruct((NKB, 1, KB), jnp.float32),
                   jax.ShapeDtypeStruct((PCD,), jnp.float32)),
    )(Kp)


def _k_scale(Kp, rs, cs):
    def body(Kp_ref, rs_ref, cs_ref, Ks_ref):
        dB = lax.rsqrt(rs_ref[...][0, 0] + 1.0)
        dA = lax.rsqrt(cs_ref[...] + 1.0)
        low, high = _unpack(Kp_ref[...])
        Ks_ref[:, 0:PC] = dB[:, None] * low * dA[None, 0:PC]
        Ks_ref[:, PC:PCD] = dB[:, None] * high * dA[None, PC:PCD]

    return pl.pallas_call(
        body,
        grid=(NKB,),
        in_specs=[pl.BlockSpec((KB, PC), lambda i: (i, 0)),
                  pl.BlockSpec((1, 1, KB), lambda i: (i, 0, 0)),
                  pl.BlockSpec((PCD,), lambda i: (0,))],
        out_specs=pl.BlockSpec((KB, PCD), lambda i: (i, 0)),
        out_shape=jax.ShapeDtypeStruct((2000, PCD), jnp.float32),
    )(Kp, rs, cs)


def _encode(xm, xp, W1):
    def body(xm_ref, xp_ref, W1_ref, hm_ref, hB_ref, hAp_ref):
        W1 = W1_ref[...]
        hm = _dot(xm_ref[...], W1)
        hm_ref[...] = hm
        hB_ref[...] = _dot(xp_ref[...], W1[0:96, :])
        h3 = hm[0:NP_].reshape(1000, 2, 128)
        hAp_ref[...] = jnp.zeros((PCD, 128), jnp.float32)
        hAp_ref[0:1000, :] = h3[:, 0, :]
        hAp_ref[PC:PC + 1000, :] = h3[:, 1, :]

    return pl.pallas_call(
        body,
        out_shape=(jax.ShapeDtypeStruct((NM_, 128), jnp.float32),
                   jax.ShapeDtypeStruct((NP_, 128), jnp.float32),
                   jax.ShapeDtypeStruct((PCD, 128), jnp.float32)),
    )(xm, xp, W1)


def _gcn_layer(Ks, hAp, hB, rs, cs, b):
    d = hAp.shape[1]

    def body(Ks_ref, hAp_ref, hB_ref, rs_ref, cs_ref, b_ref, oA_ref, oB_ref):
        i = pl.program_id(0)
        Ks = Ks_ref[...]
        hAp = hAp_ref[...]
        dB2 = 1.0 / (rs_ref[...][0, 0] + 1.0)
        b = b_ref[...][None, :]
        oB_ref[...] = _dot(Ks, hAp) + dB2[:, None] * hB_ref[...] + b

        @pl.when(i == 0)
        def _():
            dA2 = 1.0 / (cs_ref[...] + 1.0)
            oA_ref[...] = dA2[:, None] * hAp + b

        oA_ref[...] += _dot_t(Ks, hB_ref[...])

    return pl.pallas_call(
        body,
        grid=(NKB,),
        in_specs=[pl.BlockSpec((KB, PCD), lambda i: (i, 0)),
                  pl.BlockSpec((PCD, d), lambda i: (0, 0)),
                  pl.BlockSpec((KB, d), lambda i: (i, 0)),
                  pl.BlockSpec((1, 1, KB), lambda i: (i, 0, 0)),
                  pl.BlockSpec((PCD,), lambda i: (0,)),
                  pl.BlockSpec((d,), lambda i: (0,))],
        out_specs=(pl.BlockSpec((PCD, d), lambda i: (0, 0)),
                   pl.BlockSpec((KB, d), lambda i: (i, 0))),
        out_shape=(jax.ShapeDtypeStruct((PCD, d), jnp.float32),
                   jax.ShapeDtypeStruct((NP_, d), jnp.float32)),
    )(Ks, hAp, hB, rs, cs, b)


def _relu_matmul2(a, b, W):
    def body(a_ref, b_ref, W_ref, oa_ref, ob_ref):
        W = W_ref[...]
        oa_ref[...] = _dot(jnp.maximum(a_ref[...], 0.0), W)
        ob_ref[...] = _dot(jnp.maximum(b_ref[...], 0.0), W)

    d = W.shape[1]
    return pl.pallas_call(
        body,
        out_shape=(jax.ShapeDtypeStruct((a.shape[0], d), jnp.float32),
                   jax.ShapeDtypeStruct((b.shape[0], d), jnp.float32)),
    )(a, b, W)


def _decode(zAp, zB, hm, b1, W2, b2, Wd, bd):
    def body(zAp_ref, zB_ref, hm_ref, b1_ref, W2_ref, b2_ref, Wd_ref, bd_ref,
             om_ref, op_ref):
        Wd = Wd_ref[...]
        bd = bd_ref[...][None, :]
        xA_perm = _dot(zAp_ref[...], Wd) + bd
        x3 = jnp.stack([xA_perm[0:1000], xA_perm[PC:PC + 1000]], axis=1)
        om_ref[0:NP_, :] = x3.reshape(NP_, 128)
        r = jnp.maximum(hm_ref[NP_:NM_, :] + b1_ref[...][None, :], 0.0)
        z = _dot(r, W2_ref[...]) + b2_ref[...][None, :]
        om_ref[NP_:NM_, :] = _dot(z, Wd) + bd
        op_ref[...] = _dot(zB_ref[...], Wd) + bd

    return pl.pallas_call(
        body,
        out_shape=(jax.ShapeDtypeStruct((NM_, 128), jnp.float32),
                   jax.ShapeDtypeStruct((NP_, 128), jnp.float32)),
    )(zAp, zB, hm, b1, W2, b2, Wd, bd)


def kernel(x_member, x_provider, W1, b1, W2, b2, Wd, bd, edge_index):
    pidx = edge_index[0]
    midx = edge_index[1]
    k_flat = _sc_build_k(pidx, midx)
    Kp = k_flat.reshape(2000, PC)

    rs, cs = _k_sums(Kp)
    Ks = _k_scale(Kp, rs, cs)

    hm, hB, hAp = _encode(x_member, x_provider, W1)
    a1Ap, a1B = _gcn_layer(Ks, hAp, hB, rs, cs, b1)
    gAp, gB = _relu_matmul2(a1Ap, a1B, W2)
    zAp, zB = _gcn_layer(Ks, gAp, gB, rs, cs, b2)

    xhat_m, xhat_p = _decode(zAp, zB, hm, b1, W2, b2, Wd, bd)
    edge_logits = jnp.zeros((edge_index.shape[1],), dtype=jnp.float32)
    return (xhat_m, xhat_p[:, : x_provider.shape[1]], edge_logits)
